# baseline (speedup 1.0000x reference)
import jax
import jax.numpy as jnp
from jax.experimental import pallas as pl
from jax.experimental.pallas import tpu as pltpu

B = 1024
HB = B
D = 128
C = 256
CAP = 64
TAUC = 1.0
CK = 32
NCHUNK = C // CK
W = CK * CAP

_f32 = jnp.float32
_bf16 = jnp.bfloat16


def _loss_kernel(xh_ref, xb_ref, mem_ref, clsh_ref, clsf_ref, clsr_ref,
                 out_ref):
    xh = xh_ref[:, :]
    xb = xb_ref[:, :]
    cls_col = clsh_ref[:, :]
    cls_full = clsf_ref[:, :]
    cls_row = clsr_ref[:, :]

    nrm = jnp.sqrt(jnp.sum(xh * xh, axis=1, keepdims=True))
    feat = (xh / jnp.maximum(nrm, 1e-12)).astype(_bf16)

    mf32 = mem_ref[:, :].astype(_f32)
    msq = jnp.maximum(jnp.max(jnp.sum(xh * xh, axis=1, keepdims=True)),
                      jnp.max(jnp.sum(mf32 * mf32, axis=1, keepdims=True)))
    m = jnp.sqrt(msq) * 1.01 + 1e-3

    cls_iota = jax.lax.broadcasted_iota(jnp.int32, (B, C), 1)
    onehot = (cls_full == cls_iota).astype(_bf16)
    counts_row = jnp.sum(onehot.astype(_f32), axis=0, keepdims=True)

    ii = jax.lax.broadcasted_iota(jnp.int32, (B, B), 0)
    jj = jax.lax.broadcasted_iota(jnp.int32, (B, B), 1)
    lt = (jj < ii).astype(_bf16)
    cex = jax.lax.dot_general(lt, onehot, (((1,), (0,)), ((), ())),
                              preferred_element_type=_f32)
    r_col = jnp.sum(cex * onehot.astype(_f32), axis=1, keepdims=True)
    ident = (jj == ii).astype(_bf16)
    r_row = jax.lax.dot_general(r_col.astype(_bf16), ident,
                                (((0,), (0,)), ((), ())),
                                preferred_element_type=_f32)
    incl_row = r_row < float(CAP)

    same = cls_col == cls_row

    h_col = (cls_col // 64).astype(_f32)
    p_col = (cls_col - (cls_col // 64) * 64).astype(_f32)
    cm4_row = (cls_row - (cls_row // 4) * 4).astype(_f32)
    ex_new = (cm4_row == h_col) & (r_row == p_col)

    G = jax.lax.dot_general(feat, xb, (((1,), (1,)), ((), ())),
                            preferred_element_type=_f32) * (1.0 / TAUC)

    col = jax.lax.broadcasted_iota(jnp.int32, (1, W), 1)
    lcls = col // CAP
    t_f = (col - lcls * CAP).astype(_f32)
    gm4 = (lcls - (lcls // 4) * 4).astype(_f32)
    oc = (jax.lax.broadcasted_iota(jnp.int32, (W, CK), 0) // CAP ==
          jax.lax.broadcasted_iota(jnp.int32, (W, CK), 1)).astype(_f32)

    T = jnp.zeros((HB, 1), _f32)
    posM = jnp.zeros((HB, 1), _f32)

    for k in range(NCHUNK):
        mb = mem_ref[pl.ds(k * W, W), :]
        Mc = jax.lax.dot_general(feat, mb, (((1,), (1,)), ((), ())),
                                 preferred_element_type=_f32) * (1.0 / TAUC)

        countsc = counts_row[:, k * CK:(k + 1) * CK]
        ccol = jax.lax.dot_general(countsc, oc, (((1,), (1,)), ((), ())),
                                   preferred_element_type=_f32)
        tpc = t_f + ccol
        keep_b = tpc < float(CAP)

        ex_old = (gm4 == h_col) & (tpc == p_col)
        negb = jnp.logical_and(keep_b, jnp.logical_not(ex_old))
        ev = jnp.exp(jnp.where(negb, Mc, -1e4) - m)
        T = T + jnp.sum(ev, axis=1, keepdims=True)

        excl_b = (lcls + (k * CK)) == cls_col
        pv = jnp.where(jnp.logical_and(excl_b, keep_b), Mc, 0.0)
        posM = posM + jnp.sum(pv, axis=1, keepdims=True)

    negGb = jnp.logical_and(incl_row, jnp.logical_not(ex_new))
    eG = jnp.exp(jnp.where(negGb, G, -1e4) - m)
    T = T + jnp.sum(eG, axis=1, keepdims=True)
    posG = jnp.sum(jnp.where(jnp.logical_and(same, incl_row), G, 0.0),
                   axis=1, keepdims=True)

    pos = (posM + posG) * (1.0 / CAP)
    denom = jnp.exp(pos - m) + T
    lossv = (m - pos) + jnp.log(denom)
    out_ref[:, :, :] = jnp.reshape(jnp.sum(lossv), (1, 1, 1))


def kernel(x, memory, classes):
    mem_flat = memory.reshape(C * CAP, D).astype(_bf16)
    xbf = x.astype(_bf16)
    cls2d = classes.reshape(B, 1)
    cls_row = classes.reshape(1, B)
    grid = (B // HB,)
    out = pl.pallas_call(
        _loss_kernel,
        grid=grid,
        in_specs=[
            pl.BlockSpec((HB, D), lambda p: (p, 0)),
            pl.BlockSpec((B, D), lambda p: (0, 0)),
            pl.BlockSpec((C * CAP, D), lambda p: (0, 0)),
            pl.BlockSpec((HB, 1), lambda p: (p, 0)),
            pl.BlockSpec((B, 1), lambda p: (0, 0)),
            pl.BlockSpec((1, B), lambda p: (0, 0)),
        ],
        out_specs=pl.BlockSpec((1, 1, 1), lambda p: (p, 0, 0)),
        out_shape=jax.ShapeDtypeStruct((B // HB, 1, 1), jnp.float32),
        compiler_params=pltpu.CompilerParams(
            dimension_semantics=("parallel",)),
    )(x, xbf, mem_flat, cls2d, cls2d, cls_row)
    return jnp.sum(out) * (1.0 / B)


# speedup vs baseline: 1.5024x; 1.5024x over previous
import jax
import jax.numpy as jnp
from jax.experimental import pallas as pl

B = 1024
D = 128
C = 256
CAP = 64
TAUC = 1.0
CK = 32
NCHUNK = C // CK
W = CK * CAP

_f32 = jnp.float32
_bf16 = jnp.bfloat16


def _loss_kernel(x_ref, xb_ref, mem_ref, clsc_ref, out_ref):
    x = x_ref[:, :]
    xb = xb_ref[:, :]
    cls_col = clsc_ref[:, :]

    nrm = jnp.sqrt(jnp.sum(x * x, axis=1, keepdims=True))
    feat = (x / jnp.maximum(nrm, 1e-12)).astype(_bf16)

    cls_iota = jax.lax.broadcasted_iota(jnp.int32, (B, C), 1)
    onehotb = (cls_col == cls_iota).astype(_bf16)
    onehotf = onehotb.astype(_f32)
    counts_row = jnp.sum(onehotf, axis=0, keepdims=True)

    ii = jax.lax.broadcasted_iota(jnp.int32, (B, B), 0)
    jj = jax.lax.broadcasted_iota(jnp.int32, (B, B), 1)
    lt = (jj < ii).astype(_bf16)
    cex = jax.lax.dot_general(lt, onehotb, (((1,), (0,)), ((), ())),
                              preferred_element_type=_f32)
    r_col = jnp.sum(cex * onehotf, axis=1, keepdims=True).astype(jnp.int32)
    incl_col = (r_col < CAP).astype(_bf16)

    cm4_col = cls_col - (cls_col // 4) * 4
    bidx_g = jnp.where(r_col < CAP, cm4_col * CAP + r_col, C)
    gw = ((bidx_g == jax.lax.broadcasted_iota(jnp.int32, (B, C), 1))
          .astype(_bf16))
    qw = onehotb * incl_col

    G = jax.lax.dot_general(feat, xb, (((1,), (1,)), ((), ())),
                            preferred_element_type=_f32)
    Gb = G.astype(_bf16)
    eG = jnp.exp(Gb)
    TG = jax.lax.dot_general(eG, incl_col, (((1,), (0,)), ((), ())),
                             preferred_element_type=_f32)
    GEX = jax.lax.dot_general(eG, gw, (((1,), (0,)), ((), ())),
                              preferred_element_type=_f32)
    POSG = jax.lax.dot_general(Gb, qw, (((1,), (0,)), ((), ())),
                               preferred_element_type=_f32)

    colw = jax.lax.broadcasted_iota(jnp.int32, (W, 1), 0)
    lcls_w = colw // CAP
    t_w = colw - lcls_w * CAP
    gm4_w = lcls_w - (lcls_w // 4) * 4
    oc = (jax.lax.broadcasted_iota(jnp.int32, (W, CK), 0) // CAP ==
          jax.lax.broadcasted_iota(jnp.int32, (W, CK), 1))
    ocb = oc.astype(_bf16)
    ocf = oc.astype(_f32)
    kiota = jax.lax.broadcasted_iota(jnp.int32, (W, C), 1)

    T = TG
    EX = GEX
    posacc = jnp.sum(POSG * onehotf, axis=1, keepdims=True)

    for k in range(NCHUNK):
        mb = mem_ref[pl.ds(k * W, W), :]
        Mc = jax.lax.dot_general(feat, mb, (((1,), (1,)), ((), ())),
                                 preferred_element_type=_f32).astype(_bf16)
        eM = jnp.exp(Mc)

        countsc = counts_row[:, k * CK:(k + 1) * CK]
        ccol_w = jax.lax.dot_general(ocf, countsc,
                                     (((1,), (1,)), ((), ())),
                                     preferred_element_type=_f32)
        tpc_w = t_w + ccol_w.astype(jnp.int32)
        keep_w = (tpc_w < CAP).astype(_bf16)
        bidx_w = jnp.where(tpc_w < CAP, gm4_w * CAP + tpc_w, C)
        exw = (bidx_w == kiota).astype(_bf16)
        kwc = ocb * keep_w

        T = T + jax.lax.dot_general(eM, keep_w, (((1,), (0,)), ((), ())),
                                    preferred_element_type=_f32)
        EX = EX + jax.lax.dot_general(eM, exw, (((1,), (0,)), ((), ())),
                                      preferred_element_type=_f32)
        posc = jax.lax.dot_general(Mc, kwc, (((1,), (0,)), ((), ())),
                                   preferred_element_type=_f32)
        posacc = posacc + jnp.sum(
            posc * onehotf[:, k * CK:(k + 1) * CK], axis=1, keepdims=True)

    ex_i = jnp.sum(EX * onehotf, axis=1, keepdims=True)
    pos = posacc * (1.0 / CAP)
    Tn = T - ex_i
    denom = jnp.exp(pos) + Tn
    lossv = jnp.log(denom) - pos
    out_ref[:, :] = jnp.reshape(jnp.sum(lossv) * (1.0 / B), (1, 1))


def kernel(x, memory, classes):
    mem_flat = memory.reshape(C * CAP, D).astype(_bf16)
    xbf = x.astype(_bf16)
    cls2d = classes.reshape(B, 1)
    out = pl.pallas_call(
        _loss_kernel,
        out_shape=jax.ShapeDtypeStruct((1, 1), jnp.float32),
    )(x, xbf, mem_flat, cls2d)
    return out[0, 0]
